# baseline (device time: 123138 ns/iter reference)
import jax
import jax.numpy as jnp
from jax import lax
from jax.experimental import pallas as pl
from jax.experimental.pallas import tpu as pltpu

N_DEV = 8
M = 512
K = 4096
N = 8192
NB = N // N_DEV
CW = 256
NSTEP = N // CW

E4M3_MAX = 448.0


def kernel(x, w_mat):
    assert x.shape == (M, K), x.shape
    assert w_mat.shape == (K, N), w_mat.shape

    def body(x_ref, w_ref, out_ref, y_ref, xbf_ref, qsend_ref, qrecv_ref,
             amax_tx_ref, amax_ref,
             send_sems, recv_sems, amax_send_sems, amax_recv_sems):
        j = pl.program_id(0)
        me = lax.axis_index("i")

        @pl.when(j == 0)
        def _():
            xbf_ref[...] = x_ref[...].astype(jnp.bfloat16)

        y_ref[:, pl.ds(j * CW, CW)] = jnp.dot(
            xbf_ref[...], w_ref[...].astype(jnp.bfloat16),
            preferred_element_type=jnp.float32,
        )

        @pl.when(j == NSTEP - 1)
        def _comm_phase():
            bsem = pltpu.get_barrier_semaphore()
            for t in range(N_DEV):
                @pl.when(me != t)
                def _():
                    pl.semaphore_signal(
                        bsem, inc=1, device_id=(t,),
                        device_id_type=pl.DeviceIdType.MESH,
                    )
            pl.semaphore_wait(bsem, N_DEV - 1)

            local_amax = jnp.maximum(jnp.max(y_ref[...]), 0.0)
            amax_tx_ref[...] = jnp.full((1, 128), local_amax, jnp.float32)
            for t in range(N_DEV):
                @pl.when(me != t)
                def _():
                    pltpu.make_async_remote_copy(
                        src_ref=amax_tx_ref,
                        dst_ref=amax_ref.at[pl.ds(me, 1)],
                        send_sem=amax_send_sems.at[t],
                        recv_sem=amax_recv_sems.at[me],
                        device_id=(t,),
                        device_id_type=pl.DeviceIdType.MESH,
                    ).start()
            amax_ref[pl.ds(me, 1), :] = amax_tx_ref[...]
            for s in range(N_DEV):
                @pl.when(me != s)
                def _():
                    pltpu.make_async_remote_copy(
                        src_ref=amax_tx_ref,
                        dst_ref=amax_ref.at[pl.ds(s, 1)],
                        send_sem=amax_send_sems.at[s],
                        recv_sem=amax_recv_sems.at[s],
                        device_id=(s,),
                        device_id_type=pl.DeviceIdType.MESH,
                    ).wait_recv()
            g_amax = jnp.max(amax_ref[...])
            scale = g_amax / E4M3_MAX
            inv = jnp.where(g_amax > 0.0, E4M3_MAX / g_amax, 0.0)

            for t in range(N_DEV):
                chunk = jnp.maximum(y_ref[:, pl.ds(t * NB, NB)], 0.0)
                q = jnp.minimum(chunk * inv, E4M3_MAX).astype(
                    jnp.float8_e4m3fn)

                @pl.when(me == t)
                def _():
                    out_ref[pl.ds(me * M, M), :] = (
                        q.astype(jnp.float32) * scale)

                @pl.when(me != t)
                def _():
                    qsend_ref[pl.ds(t, 1)] = q[None]
                    pltpu.make_async_remote_copy(
                        src_ref=qsend_ref.at[pl.ds(t, 1)],
                        dst_ref=qrecv_ref.at[pl.ds(me, 1)],
                        send_sem=send_sems.at[t],
                        recv_sem=recv_sems.at[me],
                        device_id=(t,),
                        device_id_type=pl.DeviceIdType.MESH,
                    ).start()

            for s in range(N_DEV):
                @pl.when(me != s)
                def _():
                    pltpu.make_async_remote_copy(
                        src_ref=qsend_ref.at[pl.ds(s, 1)],
                        dst_ref=qrecv_ref.at[pl.ds(s, 1)],
                        send_sem=send_sems.at[s],
                        recv_sem=recv_sems.at[s],
                        device_id=(s,),
                        device_id_type=pl.DeviceIdType.MESH,
                    ).wait_recv()
                    out_ref[pl.ds(s * M, M), :] = (
                        qrecv_ref[s].astype(jnp.float32) * scale)

            for t in range(N_DEV):
                @pl.when(me != t)
                def _():
                    pltpu.make_async_remote_copy(
                        src_ref=amax_tx_ref,
                        dst_ref=amax_ref.at[pl.ds(me, 1)],
                        send_sem=amax_send_sems.at[t],
                        recv_sem=amax_recv_sems.at[me],
                        device_id=(t,),
                        device_id_type=pl.DeviceIdType.MESH,
                    ).wait_send()
                    pltpu.make_async_remote_copy(
                        src_ref=qsend_ref.at[pl.ds(t, 1)],
                        dst_ref=qrecv_ref.at[pl.ds(me, 1)],
                        send_sem=send_sems.at[t],
                        recv_sem=recv_sems.at[me],
                        device_id=(t,),
                        device_id_type=pl.DeviceIdType.MESH,
                    ).wait_send()

    return pl.pallas_call(
        body,
        grid=(NSTEP,),
        in_specs=[
            pl.BlockSpec((M, K), lambda j: (0, 0)),
            pl.BlockSpec((K, CW), lambda j: (0, j)),
        ],
        out_specs=pl.BlockSpec((N_DEV * M, NB), lambda j: (0, 0)),
        out_shape=jax.ShapeDtypeStruct((N_DEV * M, NB), jnp.float32),
        scratch_shapes=[
            pltpu.VMEM((M, N), jnp.float32),
            pltpu.VMEM((M, K), jnp.bfloat16),
            pltpu.VMEM((N_DEV, M, NB), jnp.float8_e4m3fn),
            pltpu.VMEM((N_DEV, M, NB), jnp.float8_e4m3fn),
            pltpu.VMEM((1, 128), jnp.float32),
            pltpu.VMEM((N_DEV, 128), jnp.float32),
            pltpu.SemaphoreType.DMA((N_DEV,)),
            pltpu.SemaphoreType.DMA((N_DEV,)),
            pltpu.SemaphoreType.DMA((N_DEV,)),
            pltpu.SemaphoreType.DMA((N_DEV,)),
        ],
        compiler_params=pltpu.CompilerParams(
            collective_id=0,
            vmem_limit_bytes=64 * 1024 * 1024,
        ),
    )(x, w_mat)


# device time: 116688 ns/iter; 1.0553x vs baseline; 1.0553x over previous
import jax
import jax.numpy as jnp
from jax import lax
from jax.experimental import pallas as pl
from jax.experimental.pallas import tpu as pltpu

N_DEV = 8
M = 512
K = 4096
N = 8192
NB = N // N_DEV
CW = 512
NSTEP = N // CW

E4M3_MAX = 448.0

_GEMM_ONLY = False


def kernel(x, w_mat):
    assert x.shape == (M, K), x.shape
    assert w_mat.shape == (K, N), w_mat.shape
    x = x.astype(jnp.bfloat16)

    def body(x_ref, w_ref, out_ref, y_ref, qsend_ref, qrecv_ref,
             amax_tx_ref, amax_ref, amax_acc_ref,
             send_sems, recv_sems, amax_send_sems, amax_recv_sems):
        j = pl.program_id(0)
        me = lax.axis_index("i")

        if not _GEMM_ONLY:
            @pl.when(j == 0)
            def _():
                bsem = pltpu.get_barrier_semaphore()
                for t in range(N_DEV):
                    @pl.when(me != t)
                    def _():
                        pl.semaphore_signal(
                            bsem, inc=1, device_id=(t,),
                            device_id_type=pl.DeviceIdType.MESH,
                        )
                pl.semaphore_wait(bsem, N_DEV - 1)

        yc = jnp.dot(
            x_ref[...], w_ref[...].astype(jnp.bfloat16),
            preferred_element_type=jnp.float32,
        )
        y_ref[:, pl.ds(j * CW, CW)] = yc
        mj = jnp.maximum(jnp.max(yc), 0.0)
        amax_acc_ref[0, 0] = jnp.where(
            j == 0, mj, jnp.maximum(amax_acc_ref[0, 0], mj))

        @pl.when(j == NSTEP - 1)
        def _comm_phase():
            if _GEMM_ONLY:
                out_ref[...] = jnp.zeros((N_DEV * M, NB), jnp.float32)
                return

            amax_tx_ref[...] = jnp.full(
                (1, 128), amax_acc_ref[0, 0], jnp.float32)
            amax_ref[0, :] = amax_tx_ref[0, :]
            for c in range(1, N_DEV):
                t = (me + c) % N_DEV
                pltpu.make_async_remote_copy(
                    src_ref=amax_tx_ref,
                    dst_ref=amax_ref.at[pl.ds(c, 1)],
                    send_sem=amax_send_sems.at[c],
                    recv_sem=amax_recv_sems.at[c],
                    device_id=(t,),
                    device_id_type=pl.DeviceIdType.MESH,
                ).start()
            for c in range(1, N_DEV):
                pltpu.make_async_remote_copy(
                    src_ref=amax_tx_ref,
                    dst_ref=amax_ref.at[pl.ds(c, 1)],
                    send_sem=amax_send_sems.at[c],
                    recv_sem=amax_recv_sems.at[c],
                    device_id=(0,),
                    device_id_type=pl.DeviceIdType.MESH,
                ).wait_recv()
            g_amax = jnp.max(amax_ref[...])
            scale = g_amax / E4M3_MAX
            inv = jnp.where(g_amax > 0.0, E4M3_MAX / g_amax, 0.0)

            for c in range(1, N_DEV):
                t = (me + c) % N_DEV
                chunk = jnp.maximum(y_ref[:, pl.ds(t * NB, NB)], 0.0)
                q = jnp.minimum(chunk * inv, E4M3_MAX).astype(
                    jnp.float8_e4m3fn)
                qsend_ref[c] = q
                pltpu.make_async_remote_copy(
                    src_ref=qsend_ref.at[pl.ds(c, 1)],
                    dst_ref=qrecv_ref.at[pl.ds(c, 1)],
                    send_sem=send_sems.at[c],
                    recv_sem=recv_sems.at[c],
                    device_id=(t,),
                    device_id_type=pl.DeviceIdType.MESH,
                ).start()

            own = jnp.maximum(y_ref[:, pl.ds(me * NB, NB)], 0.0)
            qo = jnp.minimum(own * inv, E4M3_MAX).astype(jnp.float8_e4m3fn)
            out_ref[pl.ds(me * M, M), :] = qo.astype(jnp.float32) * scale

            for c in range(1, N_DEV):
                s = (me - c) % N_DEV
                pltpu.make_async_remote_copy(
                    src_ref=qsend_ref.at[pl.ds(c, 1)],
                    dst_ref=qrecv_ref.at[pl.ds(c, 1)],
                    send_sem=send_sems.at[c],
                    recv_sem=recv_sems.at[c],
                    device_id=(0,),
                    device_id_type=pl.DeviceIdType.MESH,
                ).wait_recv()
                out_ref[pl.ds(s * M, M), :] = (
                    qrecv_ref[c].astype(jnp.float32) * scale)

            for c in range(1, N_DEV):
                pltpu.make_async_remote_copy(
                    src_ref=amax_tx_ref,
                    dst_ref=amax_ref.at[pl.ds(c, 1)],
                    send_sem=amax_send_sems.at[c],
                    recv_sem=amax_recv_sems.at[c],
                    device_id=(0,),
                    device_id_type=pl.DeviceIdType.MESH,
                ).wait_send()
                pltpu.make_async_remote_copy(
                    src_ref=qsend_ref.at[pl.ds(c, 1)],
                    dst_ref=qrecv_ref.at[pl.ds(c, 1)],
                    send_sem=send_sems.at[c],
                    recv_sem=recv_sems.at[c],
                    device_id=(0,),
                    device_id_type=pl.DeviceIdType.MESH,
                ).wait_send()

    return pl.pallas_call(
        body,
        grid=(NSTEP,),
        in_specs=[
            pl.BlockSpec((M, K), lambda j: (0, 0)),
            pl.BlockSpec((K, CW), lambda j: (0, j)),
        ],
        out_specs=pl.BlockSpec((N_DEV * M, NB), lambda j: (0, 0)),
        out_shape=jax.ShapeDtypeStruct((N_DEV * M, NB), jnp.float32),
        scratch_shapes=[
            pltpu.VMEM((M, N), jnp.float32),
            pltpu.VMEM((N_DEV, M, NB), jnp.float8_e4m3fn),
            pltpu.VMEM((N_DEV, M, NB), jnp.float8_e4m3fn),
            pltpu.VMEM((1, 128), jnp.float32),
            pltpu.VMEM((N_DEV, 128), jnp.float32),
            pltpu.SMEM((1, 1), jnp.float32),
            pltpu.SemaphoreType.DMA((N_DEV,)),
            pltpu.SemaphoreType.DMA((N_DEV,)),
            pltpu.SemaphoreType.DMA((N_DEV,)),
            pltpu.SemaphoreType.DMA((N_DEV,)),
        ],
        compiler_params=pltpu.CompilerParams(
            collective_id=None if _GEMM_ONLY else 0,
            vmem_limit_bytes=64 * 1024 * 1024,
        ),
    )(x, w_mat)


# device time: 68793 ns/iter; 1.7900x vs baseline; 1.6962x over previous
import jax
import jax.numpy as jnp
from jax import lax
from jax.experimental import pallas as pl
from jax.experimental.pallas import tpu as pltpu

N_DEV = 8
M = 512
K = 4096
N = 8192
NB = N // N_DEV
CW = 512
NSTEP = N // CW

E4M3_MAX = 448.0

_GEMM_ONLY = False
_NO_RDMA = False


def kernel(x, w_mat):
    assert x.shape == (M, K), x.shape
    assert w_mat.shape == (K, N), w_mat.shape
    x = x.astype(jnp.bfloat16)

    def body(x_ref, w_ref, out_ref, y_ref, qsend_ref, qrecv_ref,
             amax_tx_ref, amax_ref, amax_acc_ref,
             send_sems, recv_sems, amax_send_sems, amax_recv_sems):
        j = pl.program_id(0)
        me = lax.axis_index("i")

        if not _GEMM_ONLY and not _NO_RDMA:
            @pl.when(j == 0)
            def _():
                bsem = pltpu.get_barrier_semaphore()
                for t in range(N_DEV):
                    @pl.when(me != t)
                    def _():
                        pl.semaphore_signal(
                            bsem, inc=1, device_id=(t,),
                            device_id_type=pl.DeviceIdType.MESH,
                        )
                pl.semaphore_wait(bsem, N_DEV - 1)

        yc = jnp.dot(
            x_ref[...], w_ref[...].astype(jnp.bfloat16),
            preferred_element_type=jnp.float32,
        )
        y_ref[:, pl.ds(j * CW, CW)] = yc
        mj = jnp.maximum(jnp.max(yc), 0.0)
        amax_acc_ref[0, 0] = jnp.where(
            j == 0, mj, jnp.maximum(amax_acc_ref[0, 0], mj))

        @pl.when(j == NSTEP - 1)
        def _comm_phase():
            if _GEMM_ONLY:
                out_ref[...] = jnp.zeros((N_DEV * M, NB), jnp.float32)
                return

            amax_tx_ref[...] = jnp.full(
                (1, 128), amax_acc_ref[0, 0], jnp.float32)
            amax_ref[0, :] = amax_tx_ref[0, :]
            if not _NO_RDMA:
                for c in range(1, N_DEV):
                    t = (me + c) % N_DEV
                    pltpu.make_async_remote_copy(
                        src_ref=amax_tx_ref,
                        dst_ref=amax_ref.at[pl.ds(c, 1)],
                        send_sem=amax_send_sems.at[c],
                        recv_sem=amax_recv_sems.at[c],
                        device_id=(t,),
                        device_id_type=pl.DeviceIdType.MESH,
                    ).start()
                for c in range(1, N_DEV):
                    pltpu.make_async_remote_copy(
                        src_ref=amax_tx_ref,
                        dst_ref=amax_ref.at[pl.ds(c, 1)],
                        send_sem=amax_send_sems.at[c],
                        recv_sem=amax_recv_sems.at[c],
                        device_id=(0,),
                        device_id_type=pl.DeviceIdType.MESH,
                    ).wait_recv()
            g_amax = jnp.max(amax_ref[...])
            scale = g_amax / E4M3_MAX
            inv = jnp.where(g_amax > 0.0, E4M3_MAX / g_amax, 0.0)

            for c in range(1, N_DEV):
                t = (me + c) % N_DEV
                chunk = jnp.maximum(y_ref[:, pl.ds(t * NB, NB)], 0.0)
                q = jnp.minimum(chunk * inv, E4M3_MAX).astype(
                    jnp.float8_e4m3fn)
                qsend_ref[c] = q
                if not _NO_RDMA:
                    pltpu.make_async_remote_copy(
                        src_ref=qsend_ref.at[pl.ds(c, 1)],
                        dst_ref=qrecv_ref.at[pl.ds(c, 1)],
                        send_sem=send_sems.at[c],
                        recv_sem=recv_sems.at[c],
                        device_id=(t,),
                        device_id_type=pl.DeviceIdType.MESH,
                    ).start()

            own = jnp.maximum(y_ref[:, pl.ds(me * NB, NB)], 0.0)
            qo = jnp.minimum(own * inv, E4M3_MAX).astype(jnp.float8_e4m3fn)
            out_ref[pl.ds(me * M, M), :] = qo.astype(jnp.float32) * scale

            for c in range(1, N_DEV):
                s = (me - c) % N_DEV
                if not _NO_RDMA:
                    pltpu.make_async_remote_copy(
                        src_ref=qsend_ref.at[pl.ds(c, 1)],
                        dst_ref=qrecv_ref.at[pl.ds(c, 1)],
                        send_sem=send_sems.at[c],
                        recv_sem=recv_sems.at[c],
                        device_id=(0,),
                        device_id_type=pl.DeviceIdType.MESH,
                    ).wait_recv()
                src = qsend_ref if _NO_RDMA else qrecv_ref
                out_ref[pl.ds(s * M, M), :] = (
                    src[c].astype(jnp.float32) * scale)

            if _NO_RDMA:
                return
            for c in range(1, N_DEV):
                pltpu.make_async_remote_copy(
                    src_ref=amax_tx_ref,
                    dst_ref=amax_ref.at[pl.ds(c, 1)],
                    send_sem=amax_send_sems.at[c],
                    recv_sem=amax_recv_sems.at[c],
                    device_id=(0,),
                    device_id_type=pl.DeviceIdType.MESH,
                ).wait_send()
                pltpu.make_async_remote_copy(
                    src_ref=qsend_ref.at[pl.ds(c, 1)],
                    dst_ref=qrecv_ref.at[pl.ds(c, 1)],
                    send_sem=send_sems.at[c],
                    recv_sem=recv_sems.at[c],
                    device_id=(0,),
                    device_id_type=pl.DeviceIdType.MESH,
                ).wait_send()

    return pl.pallas_call(
        body,
        grid=(NSTEP,),
        in_specs=[
            pl.BlockSpec((M, K), lambda j: (0, 0)),
            pl.BlockSpec((K, CW), lambda j: (0, j)),
        ],
        out_specs=pl.BlockSpec((N_DEV * M, NB), lambda j: (0, 0)),
        out_shape=jax.ShapeDtypeStruct((N_DEV * M, NB), jnp.float32),
        scratch_shapes=[
            pltpu.VMEM((M, N), jnp.float32),
            pltpu.VMEM((N_DEV, M, NB), jnp.float8_e4m3fn),
            pltpu.VMEM((N_DEV, M, NB), jnp.float8_e4m3fn),
            pltpu.VMEM((1, 128), jnp.float32),
            pltpu.VMEM((N_DEV, 128), jnp.float32),
            pltpu.SMEM((1, 1), jnp.float32),
            pltpu.SemaphoreType.DMA((N_DEV,)),
            pltpu.SemaphoreType.DMA((N_DEV,)),
            pltpu.SemaphoreType.DMA((N_DEV,)),
            pltpu.SemaphoreType.DMA((N_DEV,)),
        ],
        compiler_params=pltpu.CompilerParams(
            collective_id=None if (_GEMM_ONLY or _NO_RDMA) else 0,
            vmem_limit_bytes=64 * 1024 * 1024,
        ),
    )(x, w_mat)
